# baseline (speedup 1.0000x reference)
import functools

import jax
import jax.numpy as jnp
from jax import lax
from jax.experimental import pallas as pl
from jax.experimental.pallas import tpu as pltpu
from jax.experimental.pallas import tpu_sc as plsc

D_MODEL = 1024
D_FF = 4096
NUM_EXPERTS = 8
CAP = 320
CPAD = 336
NTOK = 4096
NSLOT = NUM_EXPERTS * CAP
NROW = NUM_EXPERTS * CPAD
ZROW = CAP

NC, NS = 2, 16
NW = NC * NS


def _router_body(x_ref, wr_ref, br_ref, pos_in_ref, pos_out_ref, loss_ref):
    x = x_ref[...]
    wr = wr_ref[...]
    logits = jnp.dot(x, wr, preferred_element_type=jnp.float32) + br_ref[...]
    m = jnp.max(logits, axis=1, keepdims=True)
    ex = jnp.exp(logits - m)
    probs = ex / jnp.sum(ex, axis=1, keepdims=True)

    lane = lax.broadcasted_iota(jnp.int32, (NTOK, NUM_EXPERTS), 1)
    pmax = jnp.max(probs, axis=1, keepdims=True)
    top1 = jnp.min(jnp.where(probs == pmax, lane, NUM_EXPERTS),
                   axis=1, keepdims=True)
    onehot = (lane == top1).astype(jnp.float32)

    f = jnp.mean(onehot, axis=0, keepdims=True)
    p_mean = jnp.mean(probs, axis=0, keepdims=True)
    loss_ref[...] = NUM_EXPERTS * jnp.sum(f * p_mean, keepdims=True)

    a = onehot
    k = 1
    while k < NTOK:
        shifted = jnp.concatenate(
            [jnp.zeros((k, NUM_EXPERTS), jnp.float32), a[: NTOK - k]], axis=0)
        a = a + shifted
        k *= 2
    rank = (jnp.sum(onehot * a, axis=1, keepdims=True) - 1.0).astype(jnp.int32)

    valid = rank < CAP
    pos_in_ref[...] = jnp.where(valid, top1 * CAP + rank, NSLOT)
    pos_out_ref[...] = jnp.where(valid, top1 * CPAD + rank, ZROW)


def _router(x_flat, wr, br2):
    return pl.pallas_call(
        _router_body,
        out_shape=[
            jax.ShapeDtypeStruct((NTOK, 1), jnp.int32),
            jax.ShapeDtypeStruct((NTOK, 1), jnp.int32),
            jax.ShapeDtypeStruct((1, 1), jnp.float32),
        ],
    )(x_flat, wr, br2)


_TOK_PER_W = NTOK // NW
_HALF = _TOK_PER_W // 2
XG_ROWS = NSLOT + 8


def _dispatch_body(pos_hbm, x_hbm, xg_hbm, pos_v, idx_a, idx_b, rows, sem):
    wid = lax.axis_index("s") * NC + lax.axis_index("c")
    base = wid * _TOK_PER_W
    pltpu.sync_copy(pos_hbm.at[pl.ds(base, _TOK_PER_W)], pos_v)
    for i in range(_HALF // 16):
        idx_a[pl.ds(i * 16, 16)] = pos_v[pl.ds(i * 16, 16)]
        idx_b[pl.ds(i * 16, 16)] = pos_v[pl.ds(_HALF + i * 16, 16)]
    for h, idx_v in ((0, idx_a), (1, idx_b)):
        pltpu.sync_copy(x_hbm.at[pl.ds(base + h * _HALF, _HALF)], rows)
        pltpu.async_copy(rows, xg_hbm.at[idx_v], sem).wait()


def _dispatch(pos_in, x_flat):
    mesh = plsc.VectorSubcoreMesh(core_axis_name="c", subcore_axis_name="s")
    return pl.kernel(
        _dispatch_body,
        out_type=jax.ShapeDtypeStruct((XG_ROWS, D_MODEL), jnp.float32),
        mesh=mesh,
        scratch_types=[
            pltpu.VMEM((_TOK_PER_W,), jnp.int32),
            pltpu.VMEM((_HALF,), jnp.int32),
            pltpu.VMEM((_HALF,), jnp.int32),
            pltpu.VMEM((_HALF, D_MODEL), jnp.float32),
            pltpu.SemaphoreType.DMA,
        ],
        compiler_params=pltpu.CompilerParams(needs_layout_passes=False),
    )(pos_in, x_flat)


_FB = 2048
_NF = D_FF // _FB


def _ffn_body(xg_ref, w1_ref, b1_ref, w2_ref, b2_ref, out_ref):
    j = pl.program_id(1)
    xg = xg_ref[...]
    h = jnp.maximum(
        jnp.dot(xg, w1_ref[0], preferred_element_type=jnp.float32)
        + b1_ref[0], 0.0)
    part = jnp.dot(h, w2_ref[0], preferred_element_type=jnp.float32)

    @pl.when(j == 0)
    def _():
        out_ref[...] = jnp.concatenate(
            [part + b2_ref[0], jnp.zeros((CPAD - CAP, D_MODEL), jnp.float32)],
            axis=0)

    @pl.when(j > 0)
    def _():
        out_ref[pl.ds(0, CAP), :] = out_ref[pl.ds(0, CAP), :] + part


def _ffn(xg, w1, b1, w2, b2):
    return pl.pallas_call(
        _ffn_body,
        grid=(NUM_EXPERTS, _NF),
        in_specs=[
            pl.BlockSpec((CAP, D_MODEL), lambda e, j: (e, 0)),
            pl.BlockSpec((1, D_MODEL, _FB), lambda e, j: (e, 0, j)),
            pl.BlockSpec((1, 1, _FB), lambda e, j: (e, 0, j)),
            pl.BlockSpec((1, _FB, D_MODEL), lambda e, j: (e, j, 0)),
            pl.BlockSpec((1, 1, D_MODEL), lambda e, j: (e, 0, 0)),
        ],
        out_specs=pl.BlockSpec((CPAD, D_MODEL), lambda e, j: (e, 0)),
        out_shape=jax.ShapeDtypeStruct((NROW, D_MODEL), jnp.float32),
        compiler_params=pltpu.CompilerParams(
            dimension_semantics=("parallel", "arbitrary")),
    )(xg, w1, b1.reshape(NUM_EXPERTS, 1, D_FF), w2,
      b2.reshape(NUM_EXPERTS, 1, D_MODEL))


_TOK_PER_W = NTOK // NW
_HALF = _TOK_PER_W // 2


def _combine_body(pos_hbm, out_hbm, y_hbm, pos_v, idx_v, rows, sem):
    wid = lax.axis_index("s") * NC + lax.axis_index("c")
    base = wid * _TOK_PER_W
    pltpu.sync_copy(pos_hbm.at[pl.ds(base, _TOK_PER_W)], pos_v)

    def half(h, _):
        def cbody(i, _):
            idx_v[pl.ds(i * 16, 16)] = pos_v[pl.ds(h * _HALF + i * 16, 16)]
            return 0

        lax.fori_loop(0, _HALF // 16, cbody, 0)
        pltpu.async_copy(out_hbm.at[idx_v], rows, sem).wait()
        pltpu.sync_copy(rows, y_hbm.at[pl.ds(base + h * _HALF, _HALF)])
        return 0

    lax.fori_loop(0, 2, half, 0)


def _combine(pos_out, ffn_out):
    mesh = plsc.VectorSubcoreMesh(core_axis_name="c", subcore_axis_name="s")
    return pl.kernel(
        _combine_body,
        out_type=jax.ShapeDtypeStruct((NTOK, D_MODEL), jnp.float32),
        mesh=mesh,
        scratch_types=[
            pltpu.VMEM((_TOK_PER_W,), jnp.int32),
            pltpu.VMEM((_HALF,), jnp.int32),
            pltpu.VMEM((_HALF, D_MODEL), jnp.float32),
            pltpu.SemaphoreType.DMA,
        ],
        compiler_params=pltpu.CompilerParams(needs_layout_passes=False),
    )(pos_out, ffn_out)


def kernel(x, Wr, br, W1, b1, W2, b2):
    B, S, D = x.shape
    x_flat = x.reshape(NTOK, D)
    pos_in, pos_out, loss = _router(x_flat, Wr, br.reshape(1, NUM_EXPERTS))
    xg = _dispatch(pos_in.reshape(NTOK), x_flat)
    ffn_out = _ffn(xg, W1, b1, W2, b2)
    y = _combine(pos_out.reshape(NTOK), ffn_out)
    return y.reshape(B, S, D), loss.reshape(())


# speedup vs baseline: 1.8325x; 1.8325x over previous
import functools

import jax
import jax.numpy as jnp
from jax import lax
from jax.experimental import pallas as pl
from jax.experimental.pallas import tpu as pltpu
from jax.experimental.pallas import tpu_sc as plsc

D_MODEL = 1024
D_FF = 4096
NUM_EXPERTS = 8
CAP = 320
CPAD = 336
NTOK = 4096
NSLOT = NUM_EXPERTS * CAP
NROW = NUM_EXPERTS * CPAD
ZROW = CAP

NC, NS = 2, 16
NW = NC * NS


def _router_body(x_ref, wr_ref, br_ref, pos_in_ref, pos_out_ref, loss_ref):
    x = x_ref[...]
    wr = wr_ref[...]
    logits = jnp.dot(x, wr, preferred_element_type=jnp.float32) + br_ref[...]
    m = jnp.max(logits, axis=1, keepdims=True)
    ex = jnp.exp(logits - m)
    probs = ex / jnp.sum(ex, axis=1, keepdims=True)

    lane = lax.broadcasted_iota(jnp.int32, (NTOK, NUM_EXPERTS), 1)
    pmax = jnp.max(probs, axis=1, keepdims=True)
    top1 = jnp.min(jnp.where(probs == pmax, lane, NUM_EXPERTS),
                   axis=1, keepdims=True)
    onehot = (lane == top1).astype(jnp.float32)

    f = jnp.mean(onehot, axis=0, keepdims=True)
    p_mean = jnp.mean(probs, axis=0, keepdims=True)
    loss_ref[...] = NUM_EXPERTS * jnp.sum(f * p_mean, keepdims=True)

    a = onehot
    k = 1
    while k < NTOK:
        shifted = jnp.concatenate(
            [jnp.zeros((k, NUM_EXPERTS), jnp.float32), a[: NTOK - k]], axis=0)
        a = a + shifted
        k *= 2
    rank = (jnp.sum(onehot * a, axis=1, keepdims=True) - 1.0).astype(jnp.int32)

    valid = rank < CAP
    pos_in_ref[...] = jnp.where(valid, top1 * CAP + rank, NSLOT)
    tok_id = lax.broadcasted_iota(jnp.int32, (NTOK, 1), 0)
    zrow = (tok_id & 7) * CPAD + CAP + ((tok_id >> 3) & 15)
    pos_out_ref[...] = jnp.where(valid, top1 * CPAD + rank, zrow)


def _router(x_flat, wr, br2):
    return pl.pallas_call(
        _router_body,
        out_shape=[
            jax.ShapeDtypeStruct((NTOK, 1), jnp.int32),
            jax.ShapeDtypeStruct((NTOK, 1), jnp.int32),
            jax.ShapeDtypeStruct((1, 1), jnp.float32),
        ],
    )(x_flat, wr, br2)


_SLOT_PER_W = NSLOT // NW


def _dispatch_body(pos_hbm, x_hbm, xg_hbm, pos_v, idx_v, myidx, rows, sem):
    wid = lax.axis_index("s") * NC + lax.axis_index("c")
    pltpu.sync_copy(pos_hbm, pos_v)

    zeros16 = jnp.zeros((16,), jnp.int32)

    def zbody(i, _):
        idx_v[pl.ds(i * 16, 16)] = zeros16
        return 0

    lax.fori_loop(0, NSLOT // 16, zbody, 0)

    def bbody(i, _):
        pv = pos_v[pl.ds(i * 16, 16)]
        tok = lax.broadcasted_iota(jnp.int32, (16,), 0) + i * 16
        plsc.store_scatter(idx_v, [pv], tok, mask=pv < NSLOT)
        return 0

    lax.fori_loop(0, NTOK // 16, bbody, 0)

    base = wid * _SLOT_PER_W

    def cbody(i, _):
        myidx[pl.ds(i * 16, 16)] = idx_v[pl.ds(base + i * 16, 16)]
        return 0

    lax.fori_loop(0, _SLOT_PER_W // 16, cbody, 0)

    pltpu.async_copy(x_hbm.at[myidx], rows, sem).wait()
    pltpu.sync_copy(rows, xg_hbm.at[pl.ds(base, _SLOT_PER_W)])


def _dispatch(pos_in, x_flat):
    mesh = plsc.VectorSubcoreMesh(core_axis_name="c", subcore_axis_name="s")
    return pl.kernel(
        _dispatch_body,
        out_type=jax.ShapeDtypeStruct((NSLOT, D_MODEL), jnp.float32),
        mesh=mesh,
        scratch_types=[
            pltpu.VMEM((NTOK,), jnp.int32),
            pltpu.VMEM((NSLOT,), jnp.int32),
            pltpu.VMEM((_SLOT_PER_W,), jnp.int32),
            pltpu.VMEM((_SLOT_PER_W, D_MODEL), jnp.float32),
            pltpu.SemaphoreType.DMA,
        ],
        compiler_params=pltpu.CompilerParams(needs_layout_passes=False),
    )(pos_in, x_flat)


_FB = 2048
_NF = D_FF // _FB


def _ffn_body(xg_ref, w1_ref, b1_ref, w2_ref, b2_ref, out_ref):
    j = pl.program_id(1)
    xg = xg_ref[...]
    h = jnp.maximum(
        jnp.dot(xg, w1_ref[0], preferred_element_type=jnp.float32)
        + b1_ref[0], 0.0)
    part = jnp.dot(h, w2_ref[0], preferred_element_type=jnp.float32)

    @pl.when(j == 0)
    def _():
        out_ref[...] = jnp.concatenate(
            [part + b2_ref[0], jnp.zeros((CPAD - CAP, D_MODEL), jnp.float32)],
            axis=0)

    @pl.when(j > 0)
    def _():
        out_ref[pl.ds(0, CAP), :] = out_ref[pl.ds(0, CAP), :] + part


def _ffn(xg, w1, b1, w2, b2):
    return pl.pallas_call(
        _ffn_body,
        grid=(NUM_EXPERTS, _NF),
        in_specs=[
            pl.BlockSpec((CAP, D_MODEL), lambda e, j: (e, 0)),
            pl.BlockSpec((1, D_MODEL, _FB), lambda e, j: (e, 0, j)),
            pl.BlockSpec((1, 1, _FB), lambda e, j: (e, 0, j)),
            pl.BlockSpec((1, _FB, D_MODEL), lambda e, j: (e, j, 0)),
            pl.BlockSpec((1, 1, D_MODEL), lambda e, j: (e, 0, 0)),
        ],
        out_specs=pl.BlockSpec((CPAD, D_MODEL), lambda e, j: (e, 0)),
        out_shape=jax.ShapeDtypeStruct((NROW, D_MODEL), jnp.float32),
        compiler_params=pltpu.CompilerParams(
            dimension_semantics=("parallel", "arbitrary")),
    )(xg, w1, b1.reshape(NUM_EXPERTS, 1, D_FF), w2,
      b2.reshape(NUM_EXPERTS, 1, D_MODEL))


_TOK_PER_W = NTOK // NW
_HALF = _TOK_PER_W // 2


def _combine_body(pos_hbm, out_hbm, y_hbm, pos_v, idx_v, rows, sem):
    wid = lax.axis_index("s") * NC + lax.axis_index("c")
    base = wid * _TOK_PER_W
    pltpu.sync_copy(pos_hbm.at[pl.ds(base, _TOK_PER_W)], pos_v)

    def half(h, _):
        def cbody(i, _):
            idx_v[pl.ds(i * 16, 16)] = pos_v[pl.ds(h * _HALF + i * 16, 16)]
            return 0

        lax.fori_loop(0, _HALF // 16, cbody, 0)
        pltpu.async_copy(out_hbm.at[idx_v], rows, sem).wait()
        pltpu.sync_copy(rows, y_hbm.at[pl.ds(base + h * _HALF, _HALF)])
        return 0

    lax.fori_loop(0, 2, half, 0)


def _combine(pos_out, ffn_out):
    mesh = plsc.VectorSubcoreMesh(core_axis_name="c", subcore_axis_name="s")
    return pl.kernel(
        _combine_body,
        out_type=jax.ShapeDtypeStruct((NTOK, D_MODEL), jnp.float32),
        mesh=mesh,
        scratch_types=[
            pltpu.VMEM((_TOK_PER_W,), jnp.int32),
            pltpu.VMEM((_HALF,), jnp.int32),
            pltpu.VMEM((_HALF, D_MODEL), jnp.float32),
            pltpu.SemaphoreType.DMA,
        ],
        compiler_params=pltpu.CompilerParams(needs_layout_passes=False),
    )(pos_out, ffn_out)


def kernel(x, Wr, br, W1, b1, W2, b2):
    B, S, D = x.shape
    x_flat = x.reshape(NTOK, D)
    pos_in, pos_out, loss = _router(x_flat, Wr, br.reshape(1, NUM_EXPERTS))
    xg = _dispatch(pos_in.reshape(NTOK), x_flat)
    ffn_out = _ffn(xg, W1, b1, W2, b2)
    y = _combine(pos_out.reshape(NTOK), ffn_out)
    return y.reshape(B, S, D), loss.reshape(())


# speedup vs baseline: 1.8335x; 1.0005x over previous
import functools

import jax
import jax.numpy as jnp
from jax import lax
from jax.experimental import pallas as pl
from jax.experimental.pallas import tpu as pltpu
from jax.experimental.pallas import tpu_sc as plsc

D_MODEL = 1024
D_FF = 4096
NUM_EXPERTS = 8
CAP = 320
CPAD = 336
NTOK = 4096
NSLOT = NUM_EXPERTS * CAP
NROW = NUM_EXPERTS * CPAD
ZROW = CAP

NC, NS = 2, 16
NW = NC * NS


def _router_body(x_ref, wr_ref, br_ref, pos_in_ref, pos_out_ref, loss_ref):
    x = x_ref[...]
    wr = wr_ref[...]
    logits = jnp.dot(x, wr, preferred_element_type=jnp.float32) + br_ref[...]
    m = jnp.max(logits, axis=1, keepdims=True)
    ex = jnp.exp(logits - m)
    probs = ex / jnp.sum(ex, axis=1, keepdims=True)

    lane = lax.broadcasted_iota(jnp.int32, (NTOK, NUM_EXPERTS), 1)
    pmax = jnp.max(probs, axis=1, keepdims=True)
    top1 = jnp.min(jnp.where(probs == pmax, lane, NUM_EXPERTS),
                   axis=1, keepdims=True)
    onehot = (lane == top1).astype(jnp.float32)

    f = jnp.mean(onehot, axis=0, keepdims=True)
    p_mean = jnp.mean(probs, axis=0, keepdims=True)
    loss_ref[...] = NUM_EXPERTS * jnp.sum(f * p_mean, keepdims=True)

    tb = 512
    ri = lax.broadcasted_iota(jnp.int32, (tb, tb), 0)
    ci = lax.broadcasted_iota(jnp.int32, (tb, tb), 1)
    tri = jnp.where(ci <= ri, 1.0, 0.0).astype(jnp.float32)
    blocks = []
    carry = jnp.zeros((1, NUM_EXPERTS), jnp.float32)
    for b in range(NTOK // tb):
        blk = onehot[b * tb:(b + 1) * tb]
        inc_b = jnp.dot(tri, blk, preferred_element_type=jnp.float32) + carry
        carry = inc_b[tb - 1: tb, :]
        blocks.append(inc_b)
    a = jnp.concatenate(blocks, axis=0)
    rank = (jnp.sum(onehot * a, axis=1, keepdims=True) - 1.0).astype(jnp.int32)

    valid = rank < CAP
    pos_in_ref[...] = jnp.where(valid, top1 * CAP + rank, NSLOT)
    tok_id = lax.broadcasted_iota(jnp.int32, (NTOK, 1), 0)
    zrow = (tok_id & 7) * CPAD + CAP + ((tok_id >> 3) & 15)
    pos_out_ref[...] = jnp.where(valid, top1 * CPAD + rank, zrow)


def _router(x_flat, wr, br2):
    return pl.pallas_call(
        _router_body,
        out_shape=[
            jax.ShapeDtypeStruct((NTOK, 1), jnp.int32),
            jax.ShapeDtypeStruct((NTOK, 1), jnp.int32),
            jax.ShapeDtypeStruct((1, 1), jnp.float32),
        ],
    )(x_flat, wr, br2)


_SLOT_PER_W = NSLOT // NW


def _dispatch_body(pos_hbm, x_hbm, xg_hbm, pos_v, idx_v, myidx, rows, sem):
    wid = lax.axis_index("s") * NC + lax.axis_index("c")
    pltpu.sync_copy(pos_hbm, pos_v)

    zeros16 = jnp.zeros((16,), jnp.int32)

    def zbody(i, _):
        idx_v[pl.ds(i * 16, 16)] = zeros16
        return 0

    lax.fori_loop(0, NSLOT // 16, zbody, 0, unroll=8)

    def bbody(i, _):
        pv = pos_v[pl.ds(i * 16, 16)]
        tok = lax.broadcasted_iota(jnp.int32, (16,), 0) + i * 16
        plsc.store_scatter(idx_v, [pv], tok, mask=pv < NSLOT)
        return 0

    lax.fori_loop(0, NTOK // 16, bbody, 0, unroll=8)

    base = wid * _SLOT_PER_W

    def cbody(i, _):
        myidx[pl.ds(i * 16, 16)] = idx_v[pl.ds(base + i * 16, 16)]
        return 0

    lax.fori_loop(0, _SLOT_PER_W // 16, cbody, 0)

    pltpu.async_copy(x_hbm.at[myidx], rows, sem).wait()
    pltpu.sync_copy(rows, xg_hbm.at[pl.ds(base, _SLOT_PER_W)])


def _dispatch(pos_in, x_flat):
    mesh = plsc.VectorSubcoreMesh(core_axis_name="c", subcore_axis_name="s")
    return pl.kernel(
        _dispatch_body,
        out_type=jax.ShapeDtypeStruct((NSLOT, D_MODEL), jnp.float32),
        mesh=mesh,
        scratch_types=[
            pltpu.VMEM((NTOK,), jnp.int32),
            pltpu.VMEM((NSLOT,), jnp.int32),
            pltpu.VMEM((_SLOT_PER_W,), jnp.int32),
            pltpu.VMEM((_SLOT_PER_W, D_MODEL), jnp.float32),
            pltpu.SemaphoreType.DMA,
        ],
        compiler_params=pltpu.CompilerParams(needs_layout_passes=False),
    )(pos_in, x_flat)


_FB = 2048
_NF = D_FF // _FB


def _ffn_body(xg_ref, w1_ref, b1_ref, w2_ref, b2_ref, out_ref):
    j = pl.program_id(1)
    xg = xg_ref[...]
    h = jnp.maximum(
        jnp.dot(xg, w1_ref[0], preferred_element_type=jnp.float32)
        + b1_ref[0], 0.0)
    part = jnp.dot(h, w2_ref[0], preferred_element_type=jnp.float32)

    @pl.when(j == 0)
    def _():
        out_ref[...] = jnp.concatenate(
            [part + b2_ref[0], jnp.zeros((CPAD - CAP, D_MODEL), jnp.float32)],
            axis=0)

    @pl.when(j > 0)
    def _():
        out_ref[pl.ds(0, CAP), :] = out_ref[pl.ds(0, CAP), :] + part


def _ffn(xg, w1, b1, w2, b2):
    return pl.pallas_call(
        _ffn_body,
        grid=(NUM_EXPERTS, _NF),
        in_specs=[
            pl.BlockSpec((CAP, D_MODEL), lambda e, j: (e, 0)),
            pl.BlockSpec((1, D_MODEL, _FB), lambda e, j: (e, 0, j)),
            pl.BlockSpec((1, 1, _FB), lambda e, j: (e, 0, j)),
            pl.BlockSpec((1, _FB, D_MODEL), lambda e, j: (e, j, 0)),
            pl.BlockSpec((1, 1, D_MODEL), lambda e, j: (e, 0, 0)),
        ],
        out_specs=pl.BlockSpec((CPAD, D_MODEL), lambda e, j: (e, 0)),
        out_shape=jax.ShapeDtypeStruct((NROW, D_MODEL), jnp.float32),
        compiler_params=pltpu.CompilerParams(
            dimension_semantics=("parallel", "arbitrary")),
    )(xg, w1, b1.reshape(NUM_EXPERTS, 1, D_FF), w2,
      b2.reshape(NUM_EXPERTS, 1, D_MODEL))


_TOK_PER_W = NTOK // NW
_HALF = _TOK_PER_W // 2


def _combine_body(pos_hbm, out_hbm, y_hbm, pos_v, idx_v, rows, sem):
    wid = lax.axis_index("s") * NC + lax.axis_index("c")
    base = wid * _TOK_PER_W
    pltpu.sync_copy(pos_hbm.at[pl.ds(base, _TOK_PER_W)], pos_v)

    def half(h, _):
        def cbody(i, _):
            idx_v[pl.ds(i * 16, 16)] = pos_v[pl.ds(h * _HALF + i * 16, 16)]
            return 0

        lax.fori_loop(0, _HALF // 16, cbody, 0)
        pltpu.async_copy(out_hbm.at[idx_v], rows, sem).wait()
        pltpu.sync_copy(rows, y_hbm.at[pl.ds(base + h * _HALF, _HALF)])
        return 0

    lax.fori_loop(0, 2, half, 0)


def _combine(pos_out, ffn_out):
    mesh = plsc.VectorSubcoreMesh(core_axis_name="c", subcore_axis_name="s")
    return pl.kernel(
        _combine_body,
        out_type=jax.ShapeDtypeStruct((NTOK, D_MODEL), jnp.float32),
        mesh=mesh,
        scratch_types=[
            pltpu.VMEM((_TOK_PER_W,), jnp.int32),
            pltpu.VMEM((_HALF,), jnp.int32),
            pltpu.VMEM((_HALF, D_MODEL), jnp.float32),
            pltpu.SemaphoreType.DMA,
        ],
        compiler_params=pltpu.CompilerParams(needs_layout_passes=False),
    )(pos_out, ffn_out)


def kernel(x, Wr, br, W1, b1, W2, b2):
    B, S, D = x.shape
    x_flat = x.reshape(NTOK, D)
    pos_in, pos_out, loss = _router(x_flat, Wr, br.reshape(1, NUM_EXPERTS))
    xg = _dispatch(pos_in.reshape(NTOK), x_flat)
    ffn_out = _ffn(xg, W1, b1, W2, b2)
    y = _combine(pos_out.reshape(NTOK), ffn_out)
    return y.reshape(B, S, D), loss.reshape(())
